# baseline (speedup 1.0000x reference)
import functools

import jax
import jax.numpy as jnp
from jax import lax
from jax.experimental import pallas as pl
from jax.experimental.pallas import tpu as pltpu
from jax.experimental.pallas import tpu_sc as plsc

NC = 2
NS = 16
NW = NC * NS
CS = 4
LANES = 16
NBUF = 3
NB = 4


def _emb_body(ids_ref, wte_ref, wpe_ref, out_ref, *scr):
    raw_v = scr[0]
    rows = tuple(tuple(scr[1 + k * NB + b] for b in range(NB))
                 for k in range(NBUF))
    pos = scr[1 + NBUF * NB:1 + NBUF * NB + NBUF]
    sems = scr[1 + NBUF * NB + NBUF:]
    gsem = sems[0:NBUF]
    psem = sems[NBUF:2 * NBUF]
    wsem = sems[2 * NBUF:3 * NBUF]

    d = wte_ref.shape[1]
    sp = raw_v.shape[1]
    nchunk = sp // CS
    s_len = wpe_ref.shape[0]
    slices = d // LANES

    wid = lax.axis_index("s") * NC + lax.axis_index("c")
    s0 = wid * sp

    for b in range(NB):
        pltpu.sync_copy(ids_ref.at[b, pl.ds(s0, sp)], raw_v.at[b])

    def start_in(i):
        k = i % NBUF
        ghs = [
            pltpu.async_copy(
                wte_ref.at[raw_v.at[b, pl.ds(i * CS, CS)]],
                rows[k][b],
                gsem[k])
            for b in range(NB)
        ]
        ph = pltpu.async_copy(wpe_ref.at[pl.ds(s0 + i * CS, CS)], pos[k],
                              psem[k])
        return ghs, ph

    inflight = [start_in(i) for i in range(NBUF)]
    writes = [None] * NBUF

    for i in range(nchunk):
        k = i % NBUF
        ghs, ph = inflight[k]
        for gh in ghs:
            gh.wait()
        ph.wait()

        @plsc.parallel_loop(0, slices, unroll=2)
        def add_slices(m, k=k):
            off = m * LANES
            for j in range(CS):
                p = pos[k][j, pl.ds(off, LANES)]
                for b in range(NB):
                    plsc.addupdate(rows[k][b].at[j, pl.ds(off, LANES)], p)

        writes[k] = [
            pltpu.async_copy(
                rows[k][b],
                out_ref.at[pl.ds(b * s_len + s0 + i * CS, CS)],
                wsem[k])
            for b in range(NB)
        ]

        prev = i - 1
        nxt = prev + NBUF
        if prev >= 0 and nxt < nchunk:
            kp = prev % NBUF
            for wh in writes[kp]:
                wh.wait()
            inflight[kp] = start_in(nxt)

    for k in range(NBUF):
        if writes[k] is not None:
            for wh in writes[k]:
                wh.wait()


def kernel(input_ids, attention_mask, wte, wpe):
    b, s = input_ids.shape
    d = wte.shape[1]
    n = b * s
    sp = s // NW
    ids = input_ids.astype(jnp.int32)

    mesh = plsc.VectorSubcoreMesh(core_axis_name="c", subcore_axis_name="s")
    run = functools.partial(
        pl.kernel,
        mesh=mesh,
        out_type=jax.ShapeDtypeStruct((n, d), jnp.float32),
        scratch_types=(
            [pltpu.VMEM((b, sp), jnp.int32)]
            + [pltpu.VMEM((CS, d), jnp.float32)] * (NBUF * NB)
            + [pltpu.VMEM((CS, d), jnp.float32)] * NBUF
            + [pltpu.SemaphoreType.DMA] * (3 * NBUF)
        ),
    )(_emb_body)
    out = run(ids, wte, wpe)
    return (attention_mask, out.reshape(b, s, d))
